# baseline (speedup 1.0000x reference)
import functools

import jax
import jax.numpy as jnp
from jax import lax
from jax.experimental import pallas as pl
from jax.experimental.pallas import tpu as pltpu
from jax.experimental.pallas import tpu_sc as plsc

_VOCAB = 100000
_D = 64
_BATCH = 4096
_SEQ = 200
_B = _BATCH * _SEQ

_NC = 2
_NS = 16
_NW = _NC * _NS
_PER_W = _B // _NW

_IDXW = 128
_CHUNK = 512
_STREAMS = _CHUNK // _IDXW
_NCHUNK = _PER_W // _CHUNK
_NBUF = 2
_SCALE = 8.0


def _gather_body(table_hbm, idx_hbm, out_hbm, idx_v, rows_v, gsems, osems):
    wid = lax.axis_index("s") * _NC + lax.axis_index("c")
    idx_row_base = wid * (_PER_W // _IDXW)
    out_base = wid * _PER_W

    pltpu.sync_copy(idx_hbm.at[pl.ds(idx_row_base, _PER_W // _IDXW)], idx_v)

    def fire_gathers(g, b):
        copies = []
        for j in range(_STREAMS):
            copies.append(
                pltpu.async_copy(table_hbm.at[idx_v.at[g * _STREAMS + j]],
                                 rows_v.at[b].at[pl.ds(j * _IDXW, _IDXW)],
                                 gsems.at[b]))
        return copies

    out_copies = [None] * _NBUF
    gathers = fire_gathers(0, 0)
    for g in range(_NCHUNK):
        b = g % _NBUF
        nb = (g + 1) % _NBUF
        if g + 1 < _NCHUNK:
            if out_copies[nb] is not None:
                out_copies[nb].wait()
            next_gathers = fire_gathers(g + 1, nb)
        for c in gathers:
            c.wait()
        out_copies[b] = pltpu.async_copy(
            rows_v.at[b],
            out_hbm.at[pl.ds(out_base + g * _CHUNK, _CHUNK)],
            osems.at[b])
        if g + 1 < _NCHUNK:
            gathers = next_gathers
    for c in out_copies:
        if c is not None:
            c.wait()


_BBLK = 128


def _finish_body(l_ref, o_ref):
    o_ref[...] = l_ref[...].reshape(_BBLK, _SEQ, _D) * _SCALE


@jax.jit
def _sc_gather_finish(table, idx2d):
    mesh = plsc.VectorSubcoreMesh(core_axis_name="c", subcore_axis_name="s")
    flat = pl.kernel(
        _gather_body,
        out_type=jax.ShapeDtypeStruct((_B, _D), jnp.float32),
        mesh=mesh,
        scratch_types=[
            pltpu.VMEM((_PER_W // _IDXW, _IDXW), jnp.int32),
            pltpu.VMEM((_NBUF, _CHUNK, _D), jnp.float32),
            pltpu.SemaphoreType.DMA((_NBUF,)),
            pltpu.SemaphoreType.DMA((_NBUF,)),
        ],
        compiler_params=pltpu.CompilerParams(use_tc_tiling_on_sc=False),
    )(table, idx2d)
    return pl.pallas_call(
        _finish_body,
        grid=(_BATCH // _BBLK,),
        in_specs=[pl.BlockSpec((_BBLK * _SEQ, _D), lambda i: (i, 0))],
        out_specs=pl.BlockSpec((_BBLK, _SEQ, _D), lambda i: (i, 0, 0)),
        out_shape=jax.ShapeDtypeStruct((_BATCH, _SEQ, _D), jnp.float32),
    )(flat)


def kernel(x, table):
    idx2d = x.reshape(_B // _IDXW, _IDXW)
    return _sc_gather_finish(table, idx2d)


# speedup vs baseline: 1.4051x; 1.4051x over previous
import functools

import jax
import jax.numpy as jnp
from jax import lax
from jax.experimental import pallas as pl
from jax.experimental.pallas import tpu as pltpu
from jax.experimental.pallas import tpu_sc as plsc

_VOCAB = 100000
_D = 64
_BATCH = 4096
_SEQ = 200
_B = _BATCH * _SEQ

_NC = 2
_NS = 16
_NW = _NC * _NS
_PER_W = _B // _NW

_IDXW = 128
_CHUNK = 512
_STREAMS = _CHUNK // _IDXW
_NCHUNK = _PER_W // _CHUNK
_NBUF = 2
_SCALE = 8.0


def _gather_body(table_hbm, idx_hbm, out_hbm, idx_v, rows_v, gsems, osems):
    wid = lax.axis_index("s") * _NC + lax.axis_index("c")
    idx_row_base = wid * (_PER_W // _IDXW)
    out_base = wid * _PER_W

    pltpu.sync_copy(idx_hbm.at[pl.ds(idx_row_base, _PER_W // _IDXW)], idx_v)

    def fire_gathers(g, b):
        copies = []
        for j in range(_STREAMS):
            copies.append(
                pltpu.async_copy(table_hbm.at[idx_v.at[g * _STREAMS + j]],
                                 rows_v.at[b].at[pl.ds(j * _IDXW, _IDXW)],
                                 gsems.at[b]))
        return copies

    out_copies = [None] * _NBUF
    gathers = fire_gathers(0, 0)
    for g in range(_NCHUNK):
        b = g % _NBUF
        nb = (g + 1) % _NBUF
        if g + 1 < _NCHUNK:
            if out_copies[nb] is not None:
                out_copies[nb].wait()
            next_gathers = fire_gathers(g + 1, nb)
        for c in gathers:
            c.wait()
        out_copies[b] = pltpu.async_copy(
            rows_v.at[b],
            out_hbm.at[pl.ds(out_base + g * _CHUNK, _CHUNK)],
            osems.at[b])
        if g + 1 < _NCHUNK:
            gathers = next_gathers
    for c in out_copies:
        if c is not None:
            c.wait()


_TROWS = 5000


def _prescale_body(t_ref, o_ref):
    o_ref[...] = t_ref[...] * _SCALE


@jax.jit
def _sc_gather_finish(table, idx2d):
    scaled = pl.pallas_call(
        _prescale_body,
        grid=(_VOCAB // _TROWS,),
        in_specs=[pl.BlockSpec((_TROWS, _D), lambda i: (i, 0))],
        out_specs=pl.BlockSpec((_TROWS, _D), lambda i: (i, 0)),
        out_shape=jax.ShapeDtypeStruct((_VOCAB, _D), jnp.float32),
    )(table)
    mesh = plsc.VectorSubcoreMesh(core_axis_name="c", subcore_axis_name="s")
    flat = pl.kernel(
        _gather_body,
        out_type=jax.ShapeDtypeStruct((_B, _D), jnp.float32),
        mesh=mesh,
        scratch_types=[
            pltpu.VMEM((_PER_W // _IDXW, _IDXW), jnp.int32),
            pltpu.VMEM((_NBUF, _CHUNK, _D), jnp.float32),
            pltpu.SemaphoreType.DMA((_NBUF,)),
            pltpu.SemaphoreType.DMA((_NBUF,)),
        ],
        compiler_params=pltpu.CompilerParams(use_tc_tiling_on_sc=False),
    )(scaled, idx2d)
    return flat.reshape(_BATCH, _SEQ, _D)


def kernel(x, table):
    idx2d = x.reshape(_B // _IDXW, _IDXW)
    return _sc_gather_finish(table, idx2d)
